# baseline (speedup 1.0000x reference)
import functools

import jax
import jax.numpy as jnp
from jax import lax
from jax.experimental import pallas as pl
from jax.experimental.pallas import tpu as pltpu
from jax.experimental.pallas import tpu_sc as plsc

_L = 10000
_B = 4096
_NC = 2
_NS = 16
_NW = _NC * _NS
_K = 4
_CB = _B // _K
_BPW = _CB // _NW
_R = 8
_NG = _BPW // _R


def _sc_gather_chunk(idx_chunk, trans_matrix):
    mesh = plsc.VectorSubcoreMesh(core_axis_name="c", subcore_axis_name="s")

    @functools.partial(
        pl.kernel,
        mesh=mesh,
        out_type=jax.ShapeDtypeStruct((_CB, _L), jnp.float32),
        scratch_types=[
            pltpu.VMEM((_BPW + 16,), jnp.int32),
            *[pltpu.VMEM((1, _L), jnp.float32) for _ in range(_R)],
            *[pltpu.SemaphoreType.DMA for _ in range(2 * _R)],
        ],
    )
    def body(idx_hbm, table_hbm, out_hbm, idx_v, *rest):
        bufs = rest[:_R]
        gsems = rest[_R:2 * _R]
        osems = rest[2 * _R:]
        wid = lax.axis_index("s") * _NC + lax.axis_index("c")
        base = wid * _BPW

        pltpu.sync_copy(idx_hbm.at[pl.ds(base, _BPW)],
                        idx_v.at[pl.ds(0, _BPW)])

        def start_gather(row, s):
            pltpu.make_async_copy(
                table_hbm.at[pl.ds(row, 1)], bufs[s], gsems[s]).start()

        def wait_gather(s):
            pltpu.make_async_copy(
                table_hbm.at[pl.ds(0, 1)], bufs[s], gsems[s]).wait()

        def start_out(row, s):
            pltpu.make_async_copy(
                bufs[s], out_hbm.at[pl.ds(row, 1)], osems[s]).start()

        def wait_out(s):
            pltpu.make_async_copy(
                bufs[s], out_hbm.at[pl.ds(base, 1)], osems[s]).wait()

        v0 = idx_v[pl.ds(0, 16)]
        for s in range(_R):
            start_gather(v0[s], s)

        def step(q, carry):
            off = pl.multiple_of(q * 16, 8)
            vq = idx_v[pl.ds(off, 16)]
            for s in range(_R):
                wait_gather(s)
                start_out(base + q * 16 + s, s)
            for s in range(_R):
                wait_out(s)
                start_gather(vq[8 + s], s)
            for s in range(_R):
                wait_gather(s)
                start_out(base + q * 16 + 8 + s, s)
            offn = pl.multiple_of(q * 16 + 16, 8)
            vn = idx_v[pl.ds(offn, 16)]
            for s in range(_R):
                wait_out(s)

                @pl.when(q * 16 + 16 + s < _BPW)
                def _(s=s, vn=vn):
                    start_gather(vn[s], s)

            return carry

        lax.fori_loop(0, _NG // 2, step, 0)

    return body(idx_chunk, trans_matrix)


_LB = 512
_RB = 512
_NLB = -(-_L // _LB)
_NRB = _CB // _RB


def _tc_scatter_band(ot_prev, piece, band):
    def body(_, in_ref, out_ref):
        out_ref[...] = in_ref[...].T

    return pl.pallas_call(
        body,
        grid=(_NLB, _NRB),
        in_specs=[
            pl.BlockSpec(memory_space=pl.ANY),
            pl.BlockSpec((_RB, _LB), lambda j, r: (r, j)),
        ],
        out_specs=pl.BlockSpec(
            (_LB, _RB), lambda j, r, band=band: (j, band * _NRB + r)),
        out_shape=jax.ShapeDtypeStruct((_L, _B), jnp.float32),
        input_output_aliases={0: 0},
    )(ot_prev, piece)


def _tc_scatter_band0(piece):
    def body(in_ref, out_ref):
        out_ref[...] = in_ref[...].T

    return pl.pallas_call(
        body,
        grid=(_NLB, _NRB),
        in_specs=[pl.BlockSpec((_RB, _LB), lambda j, r: (r, j))],
        out_specs=pl.BlockSpec((_LB, _RB), lambda j, r: (j, r)),
        out_shape=jax.ShapeDtypeStruct((_L, _B), jnp.float32),
    )(piece)


def kernel(traj, trans_matrix):
    last_loc = traj[:, -1, 1].astype(jnp.int32)
    ot = None
    for i in range(_K):
        piece = _sc_gather_chunk(
            lax.dynamic_slice(last_loc, (_CB * i,), (_CB,)), trans_matrix)
        ot = _tc_scatter_band0(piece) if ot is None else _tc_scatter_band(
            ot, piece, i)
    return ot.T


# speedup vs baseline: 1.1439x; 1.1439x over previous
import functools

import jax
import jax.numpy as jnp
from jax import lax
from jax.experimental import pallas as pl
from jax.experimental.pallas import tpu as pltpu
from jax.experimental.pallas import tpu_sc as plsc

_L = 10000
_B = 4096
_NC = 2
_NS = 16
_NW = _NC * _NS
_K = 4
_CB = _B // _K
_BPW = _CB // _NW
_R = 8
_NG = _BPW // _R


def _sc_gather_chunk(idx_chunk, trans_matrix):
    mesh = plsc.VectorSubcoreMesh(core_axis_name="c", subcore_axis_name="s")

    @functools.partial(
        pl.kernel,
        mesh=mesh,
        out_type=jax.ShapeDtypeStruct((_CB, _L), jnp.float32),
        scratch_types=[
            pltpu.VMEM((_BPW + 16,), jnp.int32),
            *[pltpu.VMEM((1, _L), jnp.float32) for _ in range(_R)],
            *[pltpu.SemaphoreType.DMA for _ in range(2 * _R)],
        ],
    )
    def body(idx_hbm, table_hbm, out_hbm, idx_v, *rest):
        bufs = rest[:_R]
        gsems = rest[_R:2 * _R]
        osems = rest[2 * _R:]
        wid = lax.axis_index("s") * _NC + lax.axis_index("c")
        base = wid * _BPW

        pltpu.sync_copy(idx_hbm.at[pl.ds(base, _BPW)],
                        idx_v.at[pl.ds(0, _BPW)])

        def start_gather(row, s):
            pltpu.make_async_copy(
                table_hbm.at[pl.ds(row, 1)], bufs[s], gsems[s]).start()

        def wait_gather(s):
            pltpu.make_async_copy(
                table_hbm.at[pl.ds(0, 1)], bufs[s], gsems[s]).wait()

        def start_out(row, s):
            pltpu.make_async_copy(
                bufs[s], out_hbm.at[pl.ds(row, 1)], osems[s]).start()

        def wait_out(s):
            pltpu.make_async_copy(
                bufs[s], out_hbm.at[pl.ds(base, 1)], osems[s]).wait()

        v0 = idx_v[pl.ds(0, 16)]
        for s in range(_R):
            start_gather(v0[s], s)

        def step(q, carry):
            off = pl.multiple_of(q * 16, 8)
            vq = idx_v[pl.ds(off, 16)]
            for s in range(_R):
                wait_gather(s)
                start_out(base + q * 16 + s, s)
            for s in range(_R):
                wait_out(s)
                start_gather(vq[8 + s], s)
            for s in range(_R):
                wait_gather(s)
                start_out(base + q * 16 + 8 + s, s)
            offn = pl.multiple_of(q * 16 + 16, 8)
            vn = idx_v[pl.ds(offn, 16)]
            for s in range(_R):
                wait_out(s)

                @pl.when(q * 16 + 16 + s < _BPW)
                def _(s=s, vn=vn):
                    start_gather(vn[s], s)

            return carry

        lax.fori_loop(0, _NG // 2, step, 0)

    return body(idx_chunk, trans_matrix)


_LB = 2560
_RB = 256
_NLB = -(-_L // _LB)
_NRB = _CB // _RB


def _tc_scatter_band(ot_prev, piece, band):
    def body(_, in_ref, out_ref):
        out_ref[...] = in_ref[...].T

    return pl.pallas_call(
        body,
        grid=(_NLB, _NRB),
        in_specs=[
            pl.BlockSpec(memory_space=pl.ANY),
            pl.BlockSpec((_RB, _LB), lambda j, r: (r, j)),
        ],
        out_specs=pl.BlockSpec(
            (_LB, _RB), lambda j, r, band=band: (j, band * _NRB + r)),
        out_shape=jax.ShapeDtypeStruct((_L, _B), jnp.float32),
        input_output_aliases={0: 0},
    )(ot_prev, piece)


def _tc_scatter_band0(piece):
    def body(in_ref, out_ref):
        out_ref[...] = in_ref[...].T

    return pl.pallas_call(
        body,
        grid=(_NLB, _NRB),
        in_specs=[pl.BlockSpec((_RB, _LB), lambda j, r: (r, j))],
        out_specs=pl.BlockSpec((_LB, _RB), lambda j, r: (j, r)),
        out_shape=jax.ShapeDtypeStruct((_L, _B), jnp.float32),
    )(piece)


def kernel(traj, trans_matrix):
    last_loc = traj[:, -1, 1].astype(jnp.int32)
    ot = None
    for i in range(_K):
        piece = _sc_gather_chunk(
            lax.dynamic_slice(last_loc, (_CB * i,), (_CB,)), trans_matrix)
        ot = _tc_scatter_band0(piece) if ot is None else _tc_scatter_band(
            ot, piece, i)
    return ot.T
